# baseline (speedup 1.0000x reference)
import functools

import jax
import jax.numpy as jnp
from jax import lax
from jax.experimental import pallas as pl
from jax.experimental.pallas import tpu as pltpu
from jax.experimental.pallas import tpu_sc as plsc

NUM_CLASSES = 100000
D = 128
B = 16384

_info = plsc.get_sparse_core_info()
NC = _info.num_cores
NS = _info.num_subcores
L = _info.num_lanes
NW = NC * NS
ROWS_PER_W = B // NW
SUB = 128
NSUB = ROWS_PER_W // SUB
GROUPS = SUB // L

_MAGIC = 0x5F3759DF


def _newton_sqrt(x):
    xi = plsc.bitcast(x, jnp.int32)
    yi = jnp.int32(_MAGIC) - lax.shift_right_logical(xi, jnp.int32(1))
    y = plsc.bitcast(yi, jnp.float32)
    half_x = x * jnp.float32(0.5)
    for _ in range(3):
        y = y * (jnp.float32(1.5) - half_x * y * y)
    return x * y


def _sc_kernel(feat_hbm, ids_hbm, cent_hbm, out_hbm, idx_v, f_bufs, c_bufs,
               out_v, sems):
    wid = lax.axis_index("s") * NC + lax.axis_index("c")
    base = wid * ROWS_PER_W

    pltpu.sync_copy(ids_hbm.at[wid], idx_v)

    lane = lax.iota(jnp.int32, L)

    def start(sub, slot):
        row0 = base + sub * SUB
        c = pltpu.async_copy(cent_hbm.at[idx_v.at[sub]], c_bufs[slot],
                             sems[2 * slot])
        f = pltpu.async_copy(feat_hbm.at[pl.ds(row0, SUB)], f_bufs[slot],
                             sems[2 * slot + 1])
        return c, f

    pending = start(0, 0)
    for sub in range(NSUB):
        slot = sub % 2
        f_buf = f_bufs[slot]
        c_buf = c_bufs[slot]
        for h in pending:
            h.wait()
        if sub + 1 < NSUB:
            pending = start(sub + 1, 1 - slot)

        def group_body(g, _, f_buf=f_buf, c_buf=c_buf, sub=sub):
            sums = jnp.zeros((L,), jnp.float32)
            for rr in range(L):
                r = g * L + rr
                parts = []
                for k in range(D // L):
                    fv = f_buf[r, pl.ds(k * L, L)]
                    cv = c_buf[r, pl.ds(k * L, L)]
                    dv = fv - cv
                    parts.append(dv * dv)
                while len(parts) > 1:
                    parts = [parts[i] + parts[i + 1]
                             for i in range(0, len(parts), 2)]
                s = jnp.sum(parts[0])
                sums = jnp.where(lane == rr, s, sums)
            dist = _newton_sqrt(sums) * jnp.float32(1.0 / D)
            out_v[pl.ds(sub * SUB + g * L, L)] = dist
            return 0

        lax.fori_loop(0, GROUPS, group_body, 0)

    pltpu.sync_copy(out_v, out_hbm.at[pl.ds(base, ROWS_PER_W)])


@jax.jit
def kernel(features, class_ids, centers):
    ids2 = class_ids.reshape(NW, NSUB, SUB).astype(jnp.int32)
    mesh = plsc.VectorSubcoreMesh(core_axis_name="c", subcore_axis_name="s")
    run = functools.partial(
        pl.kernel,
        mesh=mesh,
        compiler_params=pltpu.CompilerParams(needs_layout_passes=False),
        out_type=jax.ShapeDtypeStruct((B,), jnp.float32),
        scratch_types=[
            pltpu.VMEM((NSUB, SUB), jnp.int32),
            [pltpu.VMEM((SUB, D), jnp.float32)] * 2,
            [pltpu.VMEM((SUB, D), jnp.float32)] * 2,
            pltpu.VMEM((ROWS_PER_W,), jnp.float32),
            [pltpu.SemaphoreType.DMA] * 4,
        ],
    )(_sc_kernel)
    out = run(features, ids2, centers)
    return out.reshape(B, 1)


# speedup vs baseline: 1.0839x; 1.0839x over previous
import functools

import jax
import jax.numpy as jnp
from jax import lax
from jax.experimental import pallas as pl
from jax.experimental.pallas import tpu as pltpu
from jax.experimental.pallas import tpu_sc as plsc

NUM_CLASSES = 100000
D = 128
B = 16384

_info = plsc.get_sparse_core_info()
NC = _info.num_cores
NS = _info.num_subcores
L = _info.num_lanes
NW = NC * NS
ROWS_PER_W = B // NW
SUB = 128
NSUB = ROWS_PER_W // SUB
GROUPS = SUB // L

_MAGIC = 0x5F3759DF


def _newton_sqrt(x):
    xi = plsc.bitcast(x, jnp.int32)
    yi = jnp.int32(_MAGIC) - lax.shift_right_logical(xi, jnp.int32(1))
    y = plsc.bitcast(yi, jnp.float32)
    half_x = x * jnp.float32(0.5)
    for _ in range(3):
        y = y * (jnp.float32(1.5) - half_x * y * y)
    return x * y


def _sc_kernel(feat_hbm, ids_hbm, cent_hbm, out_hbm, idx_v, f_bufs, c_bufs,
               out_v, sems):
    wid = lax.axis_index("s") * NC + lax.axis_index("c")
    base = wid * ROWS_PER_W

    pltpu.sync_copy(ids_hbm.at[wid], idx_v)

    lane = lax.iota(jnp.int32, L)

    def start(sub, slot):
        row0 = base + sub * SUB
        c = pltpu.async_copy(cent_hbm.at[idx_v.at[sub]], c_bufs[slot],
                             sems[2 * slot])
        f = pltpu.async_copy(feat_hbm.at[pl.ds(row0 * D, SUB * D)],
                             f_bufs[slot], sems[2 * slot + 1])
        return c, f

    pending = start(0, 0)
    for sub in range(NSUB):
        slot = sub % 2
        f_buf = f_bufs[slot]
        c_buf = c_bufs[slot]
        for h in pending:
            h.wait()
        if sub + 1 < NSUB:
            pending = start(sub + 1, 1 - slot)

        for g in range(GROUPS):
            rows = lane + jnp.int32(g * L)
            rowbase = rows * jnp.int32(D)

            def col_body(i, accs, rows=rows, rowbase=rowbase,
                         f_buf=f_buf, c_buf=c_buf):
                new = list(accs)
                for u in range(8):
                    c0 = i * 8 + u
                    cols = lax.bitwise_and(lane + c0, jnp.int32(D - 1))
                    flat = rowbase + cols
                    fv = plsc.load_gather(f_buf, [flat])
                    cv = plsc.load_gather(c_buf, [rows, cols])
                    dv = fv - cv
                    new[u % 4] = new[u % 4] + dv * dv
                return tuple(new)

            zero = jnp.zeros((L,), jnp.float32)
            a0, a1, a2, a3 = lax.fori_loop(
                0, D // 8, col_body, (zero, zero, zero, zero))
            sumsq = (a0 + a1) + (a2 + a3)
            dist = _newton_sqrt(sumsq) * jnp.float32(1.0 / D)
            out_v[pl.ds(sub * SUB + g * L, L)] = dist

    pltpu.sync_copy(out_v, out_hbm.at[pl.ds(base, ROWS_PER_W)])


@jax.jit
def kernel(features, class_ids, centers):
    ids2 = class_ids.reshape(NW, NSUB, SUB).astype(jnp.int32)
    features = features.reshape(B * D)
    mesh = plsc.VectorSubcoreMesh(core_axis_name="c", subcore_axis_name="s")
    run = functools.partial(
        pl.kernel,
        mesh=mesh,
        compiler_params=pltpu.CompilerParams(needs_layout_passes=False),
        out_type=jax.ShapeDtypeStruct((B,), jnp.float32),
        scratch_types=[
            pltpu.VMEM((NSUB, SUB), jnp.int32),
            [pltpu.VMEM((SUB * D,), jnp.float32)] * 2,
            [pltpu.VMEM((SUB, D), jnp.float32)] * 2,
            pltpu.VMEM((ROWS_PER_W,), jnp.float32),
            [pltpu.SemaphoreType.DMA] * 4,
        ],
    )(_sc_kernel)
    out = run(features, ids2, centers)
    return out.reshape(B, 1)


# speedup vs baseline: 1.1163x; 1.0300x over previous
import functools

import jax
import jax.numpy as jnp
from jax import lax
from jax.experimental import pallas as pl
from jax.experimental.pallas import tpu as pltpu
from jax.experimental.pallas import tpu_sc as plsc

NUM_CLASSES = 100000
D = 128
B = 16384

_info = plsc.get_sparse_core_info()
NC = _info.num_cores
NS = _info.num_subcores
L = _info.num_lanes
NW = NC * NS
ROWS_PER_W = B // NW
SUB = 128
NSUB = ROWS_PER_W // SUB
GROUPS = SUB // L

_MAGIC = 0x5F3759DF


def _newton_sqrt(x):
    xi = plsc.bitcast(x, jnp.int32)
    yi = jnp.int32(_MAGIC) - lax.shift_right_logical(xi, jnp.int32(1))
    y = plsc.bitcast(yi, jnp.float32)
    half_x = x * jnp.float32(0.5)
    for _ in range(3):
        y = y * (jnp.float32(1.5) - half_x * y * y)
    return x * y


def _sc_kernel(feat_hbm, ids_hbm, cent_hbm, out_hbm, idx_v, f_all, c_bufs,
               out_v, sems):
    wid = lax.axis_index("s") * NC + lax.axis_index("c")
    base = wid * ROWS_PER_W

    pltpu.sync_copy(ids_hbm.at[wid], idx_v)

    lane = lax.iota(jnp.int32, L)

    fcopy = pltpu.async_copy(feat_hbm.at[pl.ds(base * D, ROWS_PER_W * D)],
                             f_all, sems[2])
    pending = pltpu.async_copy(cent_hbm.at[idx_v.at[0]], c_bufs[0], sems[0])
    fcopy.wait()

    for sub in range(NSUB):
        slot = sub % 2
        c_buf = c_bufs[slot]
        pending.wait()
        if sub + 1 < NSUB:
            pending = pltpu.async_copy(cent_hbm.at[idx_v.at[sub + 1]],
                                       c_bufs[1 - slot], sems[1 - slot])

        for g in range(GROUPS):
            rows = lane + jnp.int32(g * L)
            rowbase = (lane + jnp.int32(sub * SUB + g * L)) * jnp.int32(D)

            def col_body(i, accs, rows=rows, rowbase=rowbase, c_buf=c_buf):
                a0, a1, a2, a3 = accs
                new = []
                for u, a in enumerate((a0, a1, a2, a3)):
                    c0 = i * 4 + u
                    cols = lax.bitwise_and(lane + c0, jnp.int32(D - 1))
                    fv = plsc.load_gather(f_all, [rowbase + cols])
                    cv = plsc.load_gather(c_buf, [rows, cols])
                    dv = fv - cv
                    new.append(a + dv * dv)
                return tuple(new)

            zero = jnp.zeros((L,), jnp.float32)
            a0, a1, a2, a3 = lax.fori_loop(
                0, D // 4, col_body, (zero, zero, zero, zero))
            sumsq = (a0 + a1) + (a2 + a3)
            dist = _newton_sqrt(sumsq) * jnp.float32(1.0 / D)
            out_v[pl.ds(sub * SUB + g * L, L)] = dist

    pltpu.sync_copy(out_v, out_hbm.at[pl.ds(base, ROWS_PER_W)])


@jax.jit
def kernel(features, class_ids, centers):
    ids2 = class_ids.reshape(NW, NSUB, SUB).astype(jnp.int32)
    features = features.reshape(B * D)
    mesh = plsc.VectorSubcoreMesh(core_axis_name="c", subcore_axis_name="s")
    run = functools.partial(
        pl.kernel,
        mesh=mesh,
        compiler_params=pltpu.CompilerParams(needs_layout_passes=False),
        out_type=jax.ShapeDtypeStruct((B,), jnp.float32),
        scratch_types=[
            pltpu.VMEM((NSUB, SUB), jnp.int32),
            pltpu.VMEM((ROWS_PER_W * D,), jnp.float32),
            [pltpu.VMEM((SUB, D), jnp.float32)] * 2,
            pltpu.VMEM((ROWS_PER_W,), jnp.float32),
            [pltpu.SemaphoreType.DMA] * 3,
        ],
    )(_sc_kernel)
    out = run(features, ids2, centers)
    return out.reshape(B, 1)


# speedup vs baseline: 1.2670x; 1.1350x over previous
import functools

import jax
import jax.numpy as jnp
from jax import lax
from jax.experimental import pallas as pl
from jax.experimental.pallas import tpu as pltpu
from jax.experimental.pallas import tpu_sc as plsc

NUM_CLASSES = 100000
D = 128
B = 16384

_info = plsc.get_sparse_core_info()
NC = _info.num_cores
NS = _info.num_subcores
L = _info.num_lanes
NW = NC * NS
ROWS_PER_W = B // NW
SUB = 128
NSUB = ROWS_PER_W // SUB
GROUPS = SUB // L

_MAGIC = 0x5F3759DF


def _newton_sqrt(x):
    xi = plsc.bitcast(x, jnp.int32)
    yi = jnp.int32(_MAGIC) - lax.shift_right_logical(xi, jnp.int32(1))
    y = plsc.bitcast(yi, jnp.float32)
    half_x = x * jnp.float32(0.5)
    for _ in range(3):
        y = y * (jnp.float32(1.5) - half_x * y * y)
    return x * y


def _sc_kernel(feat_hbm, ids_hbm, cent_hbm, out_hbm, idx_v, f_all, c_bufs,
               out_v, sems):
    wid = lax.axis_index("s") * NC + lax.axis_index("c")
    base = wid * ROWS_PER_W

    pltpu.sync_copy(ids_hbm.at[wid], idx_v)

    lane = lax.iota(jnp.int32, L)

    fcopy = pltpu.async_copy(feat_hbm.at[pl.ds(base * D, ROWS_PER_W * D)],
                             f_all, sems[2])
    pending = pltpu.async_copy(cent_hbm.at[idx_v.at[0]], c_bufs[0], sems[0])
    fcopy.wait()

    for sub in range(NSUB):
        slot = sub % 2
        c_buf = c_bufs[slot]
        pending.wait()
        if sub + 1 < NSUB:
            pending = pltpu.async_copy(cent_hbm.at[idx_v.at[sub + 1]],
                                       c_bufs[1 - slot], sems[1 - slot])

        def group_body(g, _, c_buf=c_buf, sub=sub):
            rows = lane + g * jnp.int32(L)
            rowbase = (rows + jnp.int32(sub * SUB)) * jnp.int32(D)

            def col_body(i, accs, rows=rows, rowbase=rowbase, c_buf=c_buf):
                new = list(accs)
                for u in range(16):
                    c0 = i * 16 + u
                    cols = lax.bitwise_and(lane + c0, jnp.int32(D - 1))
                    fv = plsc.load_gather(f_all, [rowbase + cols])
                    cv = plsc.load_gather(c_buf, [rows, cols])
                    dv = fv - cv
                    new[u % 8] = new[u % 8] + dv * dv
                return tuple(new)

            zero = jnp.zeros((L,), jnp.float32)
            accs = list(lax.fori_loop(0, D // 16, col_body, (zero,) * 8))
            while len(accs) > 1:
                accs = [accs[i] + accs[i + 1] for i in range(0, len(accs), 2)]
            dist = _newton_sqrt(accs[0]) * jnp.float32(1.0 / D)
            out_v[pl.ds(sub * SUB + g * L, L)] = dist
            return 0

        lax.fori_loop(0, GROUPS, group_body, 0)

    pltpu.sync_copy(out_v, out_hbm.at[pl.ds(base, ROWS_PER_W)])


@jax.jit
def kernel(features, class_ids, centers):
    ids2 = class_ids.reshape(NW, NSUB, SUB).astype(jnp.int32)
    features = features.reshape(B * D)
    mesh = plsc.VectorSubcoreMesh(core_axis_name="c", subcore_axis_name="s")
    run = functools.partial(
        pl.kernel,
        mesh=mesh,
        compiler_params=pltpu.CompilerParams(needs_layout_passes=False),
        out_type=jax.ShapeDtypeStruct((B,), jnp.float32),
        scratch_types=[
            pltpu.VMEM((NSUB, SUB), jnp.int32),
            pltpu.VMEM((ROWS_PER_W * D,), jnp.float32),
            [pltpu.VMEM((SUB, D), jnp.float32)] * 2,
            pltpu.VMEM((ROWS_PER_W,), jnp.float32),
            [pltpu.SemaphoreType.DMA] * 3,
        ],
    )(_sc_kernel)
    out = run(features, ids2, centers)
    return out.reshape(B, 1)
